# baseline (speedup 1.0000x reference)
import functools

import jax
import jax.numpy as jnp
from jax import lax
from jax.experimental import pallas as pl
from jax.experimental.pallas import tpu as pltpu
from jax.experimental.pallas import tpu_sc as plsc

TOP_K = 8


def _matmul_body(x_ref, wt_ref, out_ref):
    out_ref[...] = jnp.dot(x_ref[...], wt_ref[...],
                           preferred_element_type=jnp.float32)


def _router_logits_tc(x, wt, block_m=512):
    m, k = x.shape
    _, n = wt.shape
    return pl.pallas_call(
        _matmul_body,
        grid=(m // block_m,),
        in_specs=[
            pl.BlockSpec((block_m, k), lambda i: (i, 0)),
            pl.BlockSpec((k, n), lambda i: (0, 0)),
        ],
        out_specs=pl.BlockSpec((block_m, n), lambda i: (i, 0)),
        out_shape=jax.ShapeDtypeStruct((m, n), jnp.float32),
    )(x, wt)


def _topk_softmax_sc(logits_flat, n_tokens, n_experts):
    info = plsc.get_sparse_core_info()
    nc, ns, lanes = info.num_cores, info.num_subcores, info.num_lanes
    nw = nc * ns
    assert n_tokens % nw == 0 and n_experts == 4 * lanes
    tpw = n_tokens // nw

    mesh = plsc.VectorSubcoreMesh(core_axis_name="c", subcore_axis_name="s")

    @functools.partial(
        pl.kernel,
        out_type=(
            jax.ShapeDtypeStruct((n_tokens * TOP_K,), jnp.float32),
            jax.ShapeDtypeStruct((n_tokens * TOP_K,), jnp.int32),
        ),
        mesh=mesh,
        compiler_params=pltpu.CompilerParams(needs_layout_passes=False),
        scratch_types=[
            pltpu.VMEM((tpw * n_experts,), jnp.float32),
            pltpu.VMEM((tpw * TOP_K + lanes,), jnp.float32),
            pltpu.VMEM((tpw * TOP_K + lanes,), jnp.int32),
        ],
    )
    def k(logits_hbm, out_w_hbm, out_i_hbm, lbuf, wbuf, ibuf):
        wid = lax.axis_index("s") * nc + lax.axis_index("c")
        base = wid * tpw
        pltpu.sync_copy(logits_hbm.at[pl.ds(base * n_experts, tpw * n_experts)],
                        lbuf)
        lane = lax.iota(jnp.int32, lanes)
        lo8 = lane < TOP_K

        def body(t, carry):
            o = t * n_experts
            k0 = lbuf[pl.ds(o, lanes)]
            k1 = lbuf[pl.ds(o + 16, lanes)]
            k2 = lbuf[pl.ds(o + 32, lanes)]
            k3 = lbuf[pl.ds(o + 48, lanes)]
            s0k, s0v = plsc.sort_key_val(k0, lane, descending=True)
            s1k, s1v = plsc.sort_key_val(k1, lane + 16, descending=False)
            s2k, s2v = plsc.sort_key_val(k2, lane + 32, descending=True)
            s3k, s3v = plsc.sort_key_val(k3, lane + 48, descending=False)
            m01k = jnp.where(lo8, s0k, s1k)
            m01v = jnp.where(lo8, s0v, s1v)
            m23k = jnp.where(lo8, s2k, s3k)
            m23v = jnp.where(lo8, s2v, s3v)
            ak, av = plsc.sort_key_val(m01k, m01v, descending=True)
            bk, bv = plsc.sort_key_val(m23k, m23v, descending=False)
            fk0 = jnp.where(lo8, ak, bk)
            fv0 = jnp.where(lo8, av, bv)
            fk, fv = plsc.sort_key_val(fk0, fv0, descending=True)
            mx = jnp.max(fk)
            e = jnp.exp(fk - mx)
            s = jnp.sum(jnp.where(lo8, e, 0.0))
            w = e / s
            plsc.store_compressed(wbuf.at[pl.ds(t * TOP_K, lanes)], w, mask=lo8)
            plsc.store_compressed(ibuf.at[pl.ds(t * TOP_K, lanes)], fv, mask=lo8)
            return carry

        lax.fori_loop(0, tpw, body, 0)
        pltpu.sync_copy(wbuf.at[pl.ds(0, tpw * TOP_K)],
                        out_w_hbm.at[pl.ds(base * TOP_K, tpw * TOP_K)])
        pltpu.sync_copy(ibuf.at[pl.ds(0, tpw * TOP_K)],
                        out_i_hbm.at[pl.ds(base * TOP_K, tpw * TOP_K)])

    return k(logits_flat)


def kernel(x, W_gate):
    n_tokens = x.shape[0]
    n_experts = W_gate.shape[0]
    wt = W_gate.T
    n_chunks = 4
    tc = n_tokens // n_chunks
    logits_parts, w_parts, i_parts = [], [], []
    for c in range(n_chunks):
        logits_c = _router_logits_tc(jax.lax.slice_in_dim(x, c * tc, (c + 1) * tc), wt)
        w_c, i_c = _topk_softmax_sc(logits_c.reshape(-1), tc, n_experts)
        logits_parts.append(logits_c)
        w_parts.append(w_c.reshape(tc, TOP_K))
        i_parts.append(i_c.reshape(tc, TOP_K))
    return (jnp.concatenate(w_parts, axis=0),
            jnp.concatenate(i_parts, axis=0),
            jnp.concatenate(logits_parts, axis=0))


# speedup vs baseline: 2.1933x; 2.1933x over previous
import functools

import jax
import jax.numpy as jnp
from jax import lax
from jax.experimental import pallas as pl
from jax.experimental.pallas import tpu as pltpu
from jax.experimental.pallas import tpu_sc as plsc

TOP_K = 8


def _matmul_body(x_ref, wt_ref, out_ref):
    out_ref[...] = jnp.dot(x_ref[...], wt_ref[...],
                           preferred_element_type=jnp.float32)


def _router_logits_tc(x, wt, row_start, rows, block_m=512):
    _, k = x.shape
    _, n = wt.shape
    first_block = row_start // block_m
    return pl.pallas_call(
        _matmul_body,
        grid=(rows // block_m,),
        in_specs=[
            pl.BlockSpec((block_m, k), lambda i: (first_block + i, 0)),
            pl.BlockSpec((k, n), lambda i: (0, 0)),
        ],
        out_specs=pl.BlockSpec((block_m, n), lambda i: (i, 0)),
        out_shape=jax.ShapeDtypeStruct((rows, n), jnp.float32),
    )(x, wt)


def _topk_softmax_sc(logits_flat, n_tokens, n_experts):
    info = plsc.get_sparse_core_info()
    nc, ns, lanes = info.num_cores, info.num_subcores, info.num_lanes
    nw = nc * ns
    assert n_tokens % nw == 0 and n_experts == 4 * lanes
    tpw = n_tokens // nw

    mesh = plsc.VectorSubcoreMesh(core_axis_name="c", subcore_axis_name="s")

    @functools.partial(
        pl.kernel,
        out_type=(
            jax.ShapeDtypeStruct((n_tokens * TOP_K,), jnp.float32),
            jax.ShapeDtypeStruct((n_tokens * TOP_K,), jnp.int32),
        ),
        mesh=mesh,
        compiler_params=pltpu.CompilerParams(needs_layout_passes=False),
        scratch_types=[
            pltpu.VMEM((tpw * n_experts,), jnp.float32),
            pltpu.VMEM((tpw * TOP_K + lanes,), jnp.float32),
            pltpu.VMEM((tpw * TOP_K + lanes,), jnp.int32),
        ],
    )
    def k(logits_hbm, out_w_hbm, out_i_hbm, lbuf, wbuf, ibuf):
        wid = lax.axis_index("s") * nc + lax.axis_index("c")
        base = wid * tpw
        pltpu.sync_copy(logits_hbm.at[pl.ds(base * n_experts, tpw * n_experts)],
                        lbuf)
        lane = lax.iota(jnp.int32, lanes)
        lo8 = lane < TOP_K

        def body(t, carry):
            o = t * n_experts
            k0 = lbuf[pl.ds(o, lanes)]
            k1 = lbuf[pl.ds(o + 16, lanes)]
            k2 = lbuf[pl.ds(o + 32, lanes)]
            k3 = lbuf[pl.ds(o + 48, lanes)]
            s0k, s0v = plsc.sort_key_val(k0, lane, descending=True)
            s1k, s1v = plsc.sort_key_val(k1, lane + 16, descending=False)
            s2k, s2v = plsc.sort_key_val(k2, lane + 32, descending=True)
            s3k, s3v = plsc.sort_key_val(k3, lane + 48, descending=False)
            m01k = jnp.where(lo8, s0k, s1k)
            m01v = jnp.where(lo8, s0v, s1v)
            m23k = jnp.where(lo8, s2k, s3k)
            m23v = jnp.where(lo8, s2v, s3v)
            ak, av = plsc.sort_key_val(m01k, m01v, descending=True)
            bk, bv = plsc.sort_key_val(m23k, m23v, descending=False)
            fk0 = jnp.where(lo8, ak, bk)
            fv0 = jnp.where(lo8, av, bv)
            fk, fv = plsc.sort_key_val(fk0, fv0, descending=True)
            mx = jnp.max(fk)
            e = jnp.exp(fk - mx)
            s = jnp.sum(jnp.where(lo8, e, 0.0))
            w = e / s
            plsc.store_compressed(wbuf.at[pl.ds(t * TOP_K, lanes)], w, mask=lo8)
            plsc.store_compressed(ibuf.at[pl.ds(t * TOP_K, lanes)], fv, mask=lo8)
            return carry

        lax.fori_loop(0, tpw, body, 0)
        pltpu.sync_copy(wbuf.at[pl.ds(0, tpw * TOP_K)],
                        out_w_hbm.at[pl.ds(base * TOP_K, tpw * TOP_K)])
        pltpu.sync_copy(ibuf.at[pl.ds(0, tpw * TOP_K)],
                        out_i_hbm.at[pl.ds(base * TOP_K, tpw * TOP_K)])

    return k(logits_flat)


def kernel(x, W_gate):
    n_tokens = x.shape[0]
    n_experts = W_gate.shape[0]
    wt = W_gate.T
    n_chunks = 4
    tc = n_tokens // n_chunks
    logits_parts, w_parts, i_parts = [], [], []
    for c in range(n_chunks):
        logits_c = _router_logits_tc(x, wt, c * tc, tc)
        w_c, i_c = _topk_softmax_sc(logits_c.reshape(-1), tc, n_experts)
        logits_parts.append(logits_c)
        w_parts.append(w_c.reshape(tc, TOP_K))
        i_parts.append(i_c.reshape(tc, TOP_K))
    return (jnp.concatenate(w_parts, axis=0),
            jnp.concatenate(i_parts, axis=0),
            jnp.concatenate(logits_parts, axis=0))
